# baseline (speedup 1.0000x reference)
import functools

import jax
import jax.numpy as jnp
from jax.experimental import pallas as pl
from jax.experimental.pallas import tpu as pltpu

_BM = 512
_NBUF = 4


def _router_body(x_hbm, w_ref, o_hbm, xbuf, obuf, in_sems, out_sems,
                 *, nsteps):
    def in_copy(s, buf):
        return pltpu.make_async_copy(
            x_hbm.at[pl.ds(s * _BM, _BM), :],
            xbuf.at[buf],
            in_sems.at[buf],
        )

    def out_copy(s, buf):
        return pltpu.make_async_copy(
            obuf.at[buf],
            o_hbm.at[pl.ds(s * _BM, _BM), :],
            out_sems.at[buf],
        )

    for i in range(_NBUF):
        in_copy(i, i).start()

    def step(s, carry):
        buf = jax.lax.rem(s, _NBUF)
        in_copy(s, buf).wait()

        @pl.when(s >= _NBUF)
        def _():
            out_copy(s - _NBUF, buf).wait()

        obuf[buf] = jnp.dot(xbuf[buf], w_ref[...],
                            preferred_element_type=jnp.float32)
        out_copy(s, buf).start()

        @pl.when(s + _NBUF < nsteps)
        def _():
            in_copy(s + _NBUF, buf).start()

        return carry

    jax.lax.fori_loop(0, nsteps, step, 0)

    for i in range(_NBUF):
        out_copy(nsteps - _NBUF + i, i).wait()


def kernel(hidden_states, W):
    M, K = hidden_states.shape
    K2, N = W.shape
    assert K == K2 and M % _BM == 0
    nsteps = M // _BM
    return pl.pallas_call(
        functools.partial(_router_body, nsteps=nsteps),
        in_specs=[
            pl.BlockSpec(memory_space=pl.ANY),
            pl.BlockSpec((K, N), lambda: (0, 0)),
        ],
        out_specs=pl.BlockSpec(memory_space=pl.ANY),
        out_shape=jax.ShapeDtypeStruct((M, N), jnp.float32),
        scratch_shapes=[
            pltpu.VMEM((_NBUF, _BM, K), jnp.float32),
            pltpu.VMEM((_NBUF, _BM, N), jnp.float32),
            pltpu.SemaphoreType.DMA((_NBUF,)),
            pltpu.SemaphoreType.DMA((_NBUF,)),
        ],
    )(hidden_states, W)
